# Initial kernel scaffold; baseline (speedup 1.0000x reference)
import jax
import jax.numpy as jnp
from jax.experimental import pallas as pl


def kernel(x, token_table, pos_table):
    raise NotImplementedError("write your pallas kernel here")


# speedup vs baseline: 1.5617x; 1.5617x over previous
import functools

import jax
import jax.numpy as jnp
from jax import lax
from jax.experimental import pallas as pl
from jax.experimental.pallas import tpu as pltpu
from jax.experimental.pallas import tpu_sc as plsc

B, L, D = 1024, 200, 128
NC, NS = 2, 16
NW = NC * NS
N = B * L
R = N // NW
CH = 40
G = R // CH
LANES = 16

_mesh = plsc.VectorSubcoreMesh(core_axis_name="c", subcore_axis_name="s")


@functools.partial(
    pl.kernel,
    out_type=jax.ShapeDtypeStruct((N, D), jnp.float32),
    mesh=_mesh,
    scratch_types=[
        pltpu.VMEM((G, CH), jnp.int32),
        pltpu.VMEM((CH, D), jnp.float32),
        pltpu.VMEM((L, D), jnp.float32),
        pltpu.SemaphoreType.DMA,
    ],
)
def _emb_kernel(x_hbm, tok_hbm, pos_hbm, out_hbm, idx_v, buf, pos_v, sem):
    wid = lax.axis_index("s") * NC + lax.axis_index("c")
    base = wid * R
    pltpu.sync_copy(x_hbm.at[wid], idx_v)
    pltpu.sync_copy(pos_hbm, pos_v)

    def chunk(g, carry):
        pltpu.async_copy(tok_hbm.at[idx_v.at[g]], buf, sem).wait()
        p0 = lax.rem(g * CH, L)
        def row(i, c):
            p = p0 + i
            for j in range(D // LANES):
                sl = pl.ds(j * LANES, LANES)
                buf[i, sl] = buf[i, sl] + pos_v[p, sl]
            return c
        lax.fori_loop(0, CH, row, 0)
        pltpu.sync_copy(buf, out_hbm.at[pl.ds(base + g * CH, CH)])
        return carry

    lax.fori_loop(0, G, chunk, 0)


def kernel(x, token_table, pos_table):
    x3 = x.astype(jnp.int32).reshape(NW, G, CH)
    out = _emb_kernel(x3, token_table, pos_table)
    return out.reshape(B, L, D)


# speedup vs baseline: 6.4733x; 4.1450x over previous
import functools

import jax
import jax.numpy as jnp
from jax import lax
from jax.experimental import pallas as pl
from jax.experimental.pallas import tpu as pltpu
from jax.experimental.pallas import tpu_sc as plsc

B, L, D = 1024, 200, 128
NC, NS = 2, 16
NW = NC * NS
N = B * L
R = N // NW
CH = 80
G = R // CH
NB = 5
LANES = 16
P0 = [(b * CH) % L for b in range(NB)]

_mesh = plsc.VectorSubcoreMesh(core_axis_name="c", subcore_axis_name="s")


def _add_pos(buf, pos_v, p0):
    n1 = min(CH, L - p0)

    def make_body(buf_base, pos_base):
        def body(i2, c):
            i = i2 * 2
            for r in range(2):
                for j in range(D // LANES):
                    sl = pl.ds(j * LANES, LANES)
                    plsc.addupdate(buf.at[buf_base + i + r, sl],
                                   pos_v[pos_base + i + r, sl])
            return c
        return body

    lax.fori_loop(0, n1 // 2, make_body(0, p0), 0)
    if n1 < CH:
        lax.fori_loop(0, (CH - n1) // 2, make_body(n1, 0), 0)


@functools.partial(
    pl.kernel,
    out_type=jax.ShapeDtypeStruct((N, D), jnp.float32),
    mesh=_mesh,
    scratch_types=[
        pltpu.VMEM((G, CH), jnp.int32),
        pltpu.VMEM((L, D), jnp.float32),
    ] + [pltpu.VMEM((CH, D), jnp.float32) for _ in range(NB)] + [
        pltpu.SemaphoreType.DMA((NB,)),
        pltpu.SemaphoreType.DMA((NB,)),
    ],
)
def _emb_kernel(x_hbm, tok_hbm, pos_hbm, out_hbm, idx_v, pos_v,
                b0, b1, b2, b3, b4, gsem, ssem):
    bufs = (b0, b1, b2, b3, b4)
    wid = lax.axis_index("s") * NC + lax.axis_index("c")
    base = wid * R
    pltpu.sync_copy(x_hbm.at[wid], idx_v)
    pltpu.sync_copy(pos_hbm, pos_v)

    def visit(g, b, start_next, wait_prev_scatter):
        nb1 = (b + 1) % NB
        if start_next:
            if wait_prev_scatter:
                pltpu.make_async_copy(
                    bufs[nb1], out_hbm.at[pl.ds(0, CH)], ssem.at[nb1]).wait()
            pltpu.async_copy(
                tok_hbm.at[idx_v.at[g + 1]], bufs[nb1], gsem.at[nb1])
        pltpu.make_async_copy(
            tok_hbm.at[idx_v.at[g]], bufs[b], gsem.at[b]).wait()
        _add_pos(bufs[b], pos_v, P0[b])
        pltpu.async_copy(
            bufs[b], out_hbm.at[pl.ds(base + g * CH, CH)], ssem.at[b])

    pltpu.async_copy(tok_hbm.at[idx_v.at[0]], bufs[0], gsem.at[0])

    for b in range(NB):
        visit(b, b, start_next=True, wait_prev_scatter=(b == NB - 1))

    def group(g2, c):
        for b in range(NB):
            visit(g2 * NB + b, b, start_next=True, wait_prev_scatter=True)
        return c
    lax.fori_loop(1, G // NB - 1, group, 0)

    for b in range(NB):
        g = G - NB + b
        visit(g, b, start_next=(b != NB - 1), wait_prev_scatter=True)

    for b in range(NB):
        pltpu.make_async_copy(
            bufs[b], out_hbm.at[pl.ds(0, CH)], ssem.at[b]).wait()


def kernel(x, token_table, pos_table):
    x3 = x.astype(jnp.int32).reshape(NW, G, CH)
    out = _emb_kernel(x3, token_table, pos_table)
    return out.reshape(B, L, D)


# speedup vs baseline: 7.3550x; 1.1362x over previous
import functools

import jax
import jax.numpy as jnp
from jax import lax
from jax.experimental import pallas as pl
from jax.experimental.pallas import tpu as pltpu
from jax.experimental.pallas import tpu_sc as plsc

B, L, D = 1024, 200, 128
NC, NS = 2, 16
NW = NC * NS
N = B * L
R = N // NW
CH = 80
G = R // CH
NB = 5
LANES = 16
P0 = [(b * CH) % L for b in range(NB)]

_mesh = plsc.VectorSubcoreMesh(core_axis_name="c", subcore_axis_name="s")


def _add_pos(buf, pos_v, p0):
    n1 = min(CH, L - p0)

    def make_body(buf_base, pos_base):
        def body(i2, c):
            i = i2 * 2
            vals = [pos_v[pos_base + i + r, pl.ds(j * LANES, LANES)]
                    for r in range(2) for j in range(D // LANES)]
            k = 0
            for r in range(2):
                for j in range(D // LANES):
                    sl = pl.ds(j * LANES, LANES)
                    plsc.addupdate(buf.at[buf_base + i + r, sl], vals[k])
                    k += 1
            return c
        return body

    lax.fori_loop(0, n1 // 2, make_body(0, p0), 0)
    if n1 < CH:
        lax.fori_loop(0, (CH - n1) // 2, make_body(n1, 0), 0)


@functools.partial(
    pl.kernel,
    out_type=jax.ShapeDtypeStruct((N, D), jnp.float32),
    mesh=_mesh,
    scratch_types=[
        pltpu.VMEM((G, CH), jnp.int32),
        pltpu.VMEM((L, D), jnp.float32),
    ] + [pltpu.VMEM((CH, D), jnp.float32) for _ in range(NB)] + [
        pltpu.SemaphoreType.DMA((NB,)),
        pltpu.SemaphoreType.DMA((NB,)),
    ],
)
def _emb_kernel(x_hbm, tok_hbm, pos_hbm, out_hbm, idx_v, pos_v,
                b0, b1, b2, b3, b4, gsem, ssem):
    bufs = (b0, b1, b2, b3, b4)
    wid = lax.axis_index("s") * NC + lax.axis_index("c")
    base = wid * R
    pltpu.sync_copy(x_hbm.at[wid], idx_v)
    pltpu.sync_copy(pos_hbm, pos_v)

    def visit(g, b, start_next, wait_prev_scatter):
        nb2 = (b + 2) % NB
        if start_next:
            if wait_prev_scatter:
                pltpu.make_async_copy(
                    bufs[nb2], out_hbm.at[pl.ds(0, CH)], ssem.at[nb2]).wait()
            pltpu.async_copy(
                tok_hbm.at[idx_v.at[g + 2]], bufs[nb2], gsem.at[nb2])
        pltpu.make_async_copy(
            tok_hbm.at[idx_v.at[g]], bufs[b], gsem.at[b]).wait()
        _add_pos(bufs[b], pos_v, P0[b])
        pltpu.async_copy(
            bufs[b], out_hbm.at[pl.ds(base + g * CH, CH)], ssem.at[b])

    pltpu.async_copy(tok_hbm.at[idx_v.at[0]], bufs[0], gsem.at[0])
    pltpu.async_copy(tok_hbm.at[idx_v.at[1]], bufs[1], gsem.at[1])

    for b in range(NB):
        visit(b, b, start_next=True, wait_prev_scatter=(b + 2 >= NB))

    def group(g2, c):
        for b in range(NB):
            visit(g2 * NB + b, b, start_next=True, wait_prev_scatter=True)
        return c
    lax.fori_loop(1, G // NB - 1, group, 0)

    for b in range(NB):
        g = G - NB + b
        visit(g, b, start_next=(g + 2 < G), wait_prev_scatter=True)

    for b in range(NB):
        pltpu.make_async_copy(
            bufs[b], out_hbm.at[pl.ds(0, CH)], ssem.at[b]).wait()


def kernel(x, token_table, pos_table):
    x3 = x.astype(jnp.int32).reshape(NW, G, CH)
    out = _emb_kernel(x3, token_table, pos_table)
    return out.reshape(B, L, D)
